# baseline (speedup 1.0000x reference)
import functools

import jax
import jax.numpy as jnp
from jax import lax
from jax.experimental import pallas as pl
from jax.experimental.pallas import tpu as pltpu
from jax.experimental.pallas import tpu_sc as plsc

_MASKING_RATIO = 0.75

_NC = 2
_NS = 16
_NW = _NC * _NS


@functools.lru_cache(maxsize=None)
def _make_gather(n_rows: int, d: int, n_restore: int):
    assert n_rows % (8 * _NW) == 0
    rows_per_w = n_rows // _NW
    chunk = min(rows_per_w, 32)
    assert rows_per_w % chunk == 0
    n_chunks = rows_per_w // chunk
    nbuf = min(n_chunks, 3)
    assert n_restore % (8 * _NW) == 0
    res_per_w = n_restore // _NW

    mesh = plsc.VectorSubcoreMesh(core_axis_name="c", subcore_axis_name="s")

    @functools.partial(
        pl.kernel,
        mesh=mesh,
        out_type=(
            jax.ShapeDtypeStruct((n_rows, d), jnp.float32),
            jax.ShapeDtypeStruct((n_restore,), jnp.int32),
        ),
        scratch_types=[
            pltpu.VMEM((rows_per_w,), jnp.int32),
            pltpu.VMEM((res_per_w,), jnp.int32),
            [pltpu.VMEM((chunk, d), jnp.float32) for _ in range(nbuf)],
            [pltpu.SemaphoreType.DMA for _ in range(nbuf)],
            [pltpu.SemaphoreType.DMA for _ in range(nbuf)],
            pltpu.SemaphoreType.DMA,
        ],
    )
    def gather_rows(table_hbm, idx_hbm, res_hbm, rows_out, res_out,
                    idx_v, res_v, bufs, rsems, wsems, res_sem):
        wid = lax.axis_index("s") * _NC + lax.axis_index("c")
        base = wid * rows_per_w
        pltpu.sync_copy(idx_hbm.at[pl.ds(base, rows_per_w)], idx_v)
        res_base = wid * res_per_w
        res_in = pltpu.async_copy(res_hbm.at[pl.ds(res_base, res_per_w)],
                                  res_v, res_sem)

        def start_read(ci):
            return pltpu.async_copy(
                table_hbm.at[idx_v.at[pl.ds(ci * chunk, chunk)]],
                bufs[ci % nbuf],
                rsems[ci % nbuf],
            )

        reads = {ci: start_read(ci) for ci in range(min(nbuf, n_chunks))}
        writes = {}
        for ci in range(n_chunks):
            reads[ci].wait()
            writes[ci] = pltpu.async_copy(
                bufs[ci % nbuf],
                rows_out.at[pl.ds(base + ci * chunk, chunk)],
                wsems[ci % nbuf],
            )
            nxt = ci + nbuf
            if nxt < n_chunks:
                writes[ci].wait()
                reads[nxt] = start_read(nxt)
        res_in.wait()
        pltpu.async_copy(res_v, res_out.at[pl.ds(res_base, res_per_w)],
                         res_sem).wait()
        for ci in range(max(0, n_chunks - nbuf), n_chunks):
            writes[ci].wait()

    return gather_rows


@functools.lru_cache(maxsize=None)
def _mask_constants(b: int, s: int):
    import numpy as np

    with jax.ensure_compile_time_eval():
        rkey = jax.random.key(42)
        random_numbers = jax.random.normal(rkey, (b, s), dtype=jnp.float32)
        indices = np.asarray(jnp.argsort(random_numbers, axis=1))
        indices_restore = np.asarray(jnp.argsort(jnp.asarray(indices), axis=1))
    len_keep = int((1.0 - _MASKING_RATIO) * s)
    keep = indices[:, :len_keep].astype(np.int32)
    idx_flat = (keep + (np.arange(b, dtype=np.int32) * s)[:, None]).reshape(-1)
    return idx_flat, indices_restore


def kernel(x):
    b, s, d = x.shape
    idx_flat, indices_restore = _mask_constants(b, s)
    len_keep = idx_flat.shape[0] // b

    table = x.reshape(b * s, d)
    out, restore = _make_gather(b * len_keep, d, b * s)(
        table,
        jnp.asarray(idx_flat),
        jnp.asarray(indices_restore.reshape(-1), dtype=jnp.int32),
    )
    return (out.reshape(b, len_keep, d), restore.reshape(b, s))


# speedup vs baseline: 1.0222x; 1.0222x over previous
import functools

import jax
import jax.numpy as jnp
from jax import lax
from jax.experimental import pallas as pl
from jax.experimental.pallas import tpu as pltpu
from jax.experimental.pallas import tpu_sc as plsc

_MASKING_RATIO = 0.75

_NC = 2
_NS = 16
_NW = _NC * _NS


@functools.lru_cache(maxsize=None)
def _make_gather(n_rows: int, d: int):
    assert n_rows % (8 * _NW) == 0
    rows_per_w = n_rows // _NW
    chunk = min(rows_per_w, 32)
    assert rows_per_w % chunk == 0
    n_chunks = rows_per_w // chunk
    nbuf = min(n_chunks, 3)

    mesh = plsc.VectorSubcoreMesh(core_axis_name="c", subcore_axis_name="s")

    @functools.partial(
        pl.kernel,
        mesh=mesh,
        out_type=jax.ShapeDtypeStruct((n_rows, d), jnp.float32),
        scratch_types=[
            pltpu.VMEM((rows_per_w,), jnp.int32),
            [pltpu.VMEM((chunk, d), jnp.float32) for _ in range(nbuf)],
            [pltpu.SemaphoreType.DMA for _ in range(nbuf)],
            [pltpu.SemaphoreType.DMA for _ in range(nbuf)],
        ],
    )
    def gather_rows(table_hbm, idx_hbm, out_hbm, idx_v, bufs, rsems, wsems):
        wid = lax.axis_index("s") * _NC + lax.axis_index("c")
        base = wid * rows_per_w
        pltpu.sync_copy(idx_hbm.at[pl.ds(base, rows_per_w)], idx_v)

        def start_read(ci):
            return pltpu.async_copy(
                table_hbm.at[idx_v.at[pl.ds(ci * chunk, chunk)]],
                bufs[ci % nbuf],
                rsems[ci % nbuf],
            )

        reads = {ci: start_read(ci) for ci in range(min(nbuf, n_chunks))}
        writes = {}
        for ci in range(n_chunks):
            reads[ci].wait()
            writes[ci] = pltpu.async_copy(
                bufs[ci % nbuf],
                out_hbm.at[pl.ds(base + ci * chunk, chunk)],
                wsems[ci % nbuf],
            )
            nxt = ci + nbuf
            if nxt < n_chunks:
                writes[ci].wait()
                reads[nxt] = start_read(nxt)
        for ci in range(max(0, n_chunks - nbuf), n_chunks):
            writes[ci].wait()

    return gather_rows


@functools.lru_cache(maxsize=None)
def _mask_constants(b: int, s: int):
    import numpy as np

    with jax.ensure_compile_time_eval():
        rkey = jax.random.key(42)
        random_numbers = jax.random.normal(rkey, (b, s), dtype=jnp.float32)
        indices = np.asarray(jnp.argsort(random_numbers, axis=1))
        indices_restore = np.asarray(jnp.argsort(jnp.asarray(indices), axis=1))
    len_keep = int((1.0 - _MASKING_RATIO) * s)
    keep = indices[:, :len_keep].astype(np.int32)
    idx_flat = (keep + (np.arange(b, dtype=np.int32) * s)[:, None]).reshape(-1)
    return idx_flat, indices_restore


def kernel(x):
    b, s, d = x.shape
    idx_flat, indices_restore = _mask_constants(b, s)
    len_keep = idx_flat.shape[0] // b

    table = x.reshape(b * s, d)
    out = _make_gather(b * len_keep, d)(table, jnp.asarray(idx_flat))
    return (out.reshape(b, len_keep, d), jnp.asarray(indices_restore))


# speedup vs baseline: 1.0322x; 1.0097x over previous
import functools

import jax
import jax.numpy as jnp
from jax import lax
from jax.experimental import pallas as pl
from jax.experimental.pallas import tpu as pltpu
from jax.experimental.pallas import tpu_sc as plsc

_MASKING_RATIO = 0.75

_NC = 2
_NS = 16
_NW = _NC * _NS


@functools.lru_cache(maxsize=None)
def _make_gather(n_rows: int, d: int):
    assert n_rows % (8 * _NW) == 0
    rows_per_w = n_rows // _NW
    chunk = min(rows_per_w, 32)
    assert rows_per_w % chunk == 0
    n_chunks = rows_per_w // chunk
    nbuf = min(n_chunks, 3)

    mesh = plsc.VectorSubcoreMesh(core_axis_name="c", subcore_axis_name="s")

    @functools.partial(
        pl.kernel,
        mesh=mesh,
        out_type=jax.ShapeDtypeStruct((n_rows, d), jnp.float32),
        scratch_types=[
            pltpu.VMEM((rows_per_w,), jnp.int32),
            [pltpu.VMEM((chunk, d), jnp.float32) for _ in range(nbuf)],
            [pltpu.SemaphoreType.DMA for _ in range(nbuf)],
            [pltpu.SemaphoreType.DMA for _ in range(nbuf)],
        ],
    )
    def gather_rows(table_hbm, idx_hbm, out_hbm, idx_v, bufs, rsems, wsems):
        wid = lax.axis_index("s") * _NC + lax.axis_index("c")
        base = wid * rows_per_w
        pltpu.sync_copy(idx_hbm.at[pl.ds(base, rows_per_w)], idx_v)

        def start_read(ci):
            return pltpu.async_copy(
                table_hbm.at[idx_v.at[pl.ds(ci * chunk, chunk)]],
                bufs[ci % nbuf],
                rsems[ci % nbuf],
            )

        reads = {ci: start_read(ci) for ci in range(min(nbuf, n_chunks))}
        writes = {}
        for ci in range(n_chunks):
            reads[ci].wait()
            writes[ci] = pltpu.async_copy(
                bufs[ci % nbuf],
                out_hbm.at[pl.ds(base + ci * chunk, chunk)],
                wsems[ci % nbuf],
            )
            nxt = ci + nbuf
            if nxt < n_chunks:
                writes[ci].wait()
                reads[nxt] = start_read(nxt)
        for ci in range(max(0, n_chunks - nbuf), n_chunks):
            writes[ci].wait()

    return gather_rows


@functools.lru_cache(maxsize=None)
def _mask_constants(b: int, s: int):
    import numpy as np

    with jax.ensure_compile_time_eval():
        rkey = jax.random.key(42)
        random_numbers = jax.random.normal(rkey, (b, s), dtype=jnp.float32)
        indices = np.asarray(jnp.argsort(random_numbers, axis=1))
    len_keep = int((1.0 - _MASKING_RATIO) * s)
    keep = indices[:, :len_keep].astype(np.int32)
    idx_flat = (keep + (np.arange(b, dtype=np.int32) * s)[:, None]).reshape(-1)
    return idx_flat, indices


def kernel(x):
    b, s, d = x.shape
    idx_flat, indices = _mask_constants(b, s)
    len_keep = idx_flat.shape[0] // b

    table = x.reshape(b * s, d)
    out = _make_gather(b * len_keep, d)(table, jnp.asarray(idx_flat))
    indices_restore = jnp.argsort(jnp.asarray(indices), axis=1)
    return (out.reshape(b, len_keep, d), indices_restore)


# speedup vs baseline: 1.0468x; 1.0142x over previous
import functools

import jax
import jax.numpy as jnp
from jax import lax
from jax.experimental import pallas as pl
from jax.experimental.pallas import tpu as pltpu
from jax.experimental.pallas import tpu_sc as plsc

_MASKING_RATIO = 0.75

_NC = 2
_NS = 16
_NW = _NC * _NS


@functools.lru_cache(maxsize=None)
def _make_gather(n_rows: int, d: int):
    assert n_rows % (8 * _NW) == 0
    rows_per_w = n_rows // _NW
    chunk = min(rows_per_w, 32)
    assert rows_per_w % chunk == 0
    n_chunks = rows_per_w // chunk
    nbuf = min(n_chunks, 3)

    mesh = plsc.VectorSubcoreMesh(core_axis_name="c", subcore_axis_name="s")

    @functools.partial(
        pl.kernel,
        mesh=mesh,
        out_type=jax.ShapeDtypeStruct((n_rows, d), jnp.float32),
        scratch_types=[
            pltpu.VMEM((rows_per_w,), jnp.int32),
            [pltpu.VMEM((chunk, d), jnp.float32) for _ in range(nbuf)],
            [pltpu.SemaphoreType.DMA for _ in range(nbuf)],
            [pltpu.SemaphoreType.DMA for _ in range(nbuf)],
        ],
    )
    def gather_rows(table_hbm, idx_hbm, out_hbm, idx_v, bufs, rsems, wsems):
        wid = lax.axis_index("s") * _NC + lax.axis_index("c")
        base = wid * rows_per_w
        pltpu.sync_copy(idx_hbm.at[pl.ds(base, rows_per_w)], idx_v)

        def start_read(ci):
            return pltpu.async_copy(
                table_hbm.at[idx_v.at[pl.ds(ci * chunk, chunk)]],
                bufs[ci % nbuf],
                rsems[ci % nbuf],
            )

        reads = {ci: start_read(ci) for ci in range(min(nbuf, n_chunks))}
        writes = {}
        for ci in range(n_chunks):
            reads[ci].wait()
            writes[ci] = pltpu.async_copy(
                bufs[ci % nbuf],
                out_hbm.at[pl.ds(base + ci * chunk, chunk)],
                wsems[ci % nbuf],
            )
            nxt = ci + nbuf
            if nxt < n_chunks:
                writes[ci].wait()
                reads[nxt] = start_read(nxt)
        for ci in range(max(0, n_chunks - nbuf), n_chunks):
            writes[ci].wait()

    return gather_rows


@functools.lru_cache(maxsize=None)
def _mask_constants(b: int, s: int):
    import numpy as np

    with jax.ensure_compile_time_eval():
        rkey = jax.random.key(42)
        random_numbers = jax.random.normal(rkey, (b, s), dtype=jnp.float32)
        indices = np.asarray(jnp.argsort(random_numbers, axis=1))
    len_keep = int((1.0 - _MASKING_RATIO) * s)
    keep = indices[:, :len_keep].astype(np.int32)
    idx_flat = (keep + (np.arange(b, dtype=np.int32) * s)[:, None]).reshape(-1)
    return idx_flat, indices


def kernel(x):
    b, s, d = x.shape
    idx_flat, indices = _mask_constants(b, s)
    len_keep = idx_flat.shape[0] // b

    table = x.reshape(b * s, d)
    out = _make_gather(b * len_keep, d)(table, jnp.asarray(idx_flat))
    indices_restore = jnp.argsort(jnp.asarray(indices), axis=1)
    zero = (x[0, 0, 0] * 0.0).astype(jnp.int32)
    return (out.reshape(b, len_keep, d), indices_restore + zero)
